# Initial kernel scaffold; baseline (speedup 1.0000x reference)
import jax
import jax.numpy as jnp
from jax.experimental import pallas as pl


def kernel(x, edge_index, glove, W1, b1, W2, b2, W3, b3):
    raise NotImplementedError("write your pallas kernel here")


# speedup vs baseline: 23.0160x; 23.0160x over previous
import functools

import jax
import jax.numpy as jnp
from jax import lax
from jax.experimental import pallas as pl
from jax.experimental.pallas import tpu as pltpu
from jax.experimental.pallas import tpu_sc as plsc

N = 10000
NP = 10240
E = 320000
B = 128
NW = 32
EPAD = 2560 * B
NCHUNK = EPAD // B // NW
GRP = 8
NGRP = NCHUNK // GRP
RPT = NP // 16
NC = 2


def _mesh():
    return plsc.VectorSubcoreMesh(core_axis_name="c", subcore_axis_name="s")


_SC_PARAMS = pltpu.CompilerParams(use_tc_tiling_on_sc=False)


def _wid():
    return lax.axis_index("s") * NC + lax.axis_index("c")


def _degree_pass(dst2d, ones, zeros16):
    def body(dst_hbm, ones_hbm, z_hbm, out_hbm, idx_v, ones_v, acc, sem):
        cid = lax.axis_index("c")
        sid = lax.axis_index("s")
        wid = _wid()
        pltpu.sync_copy(dst_hbm.at[pl.ds(wid * NCHUNK, NCHUNK)], idx_v)
        pltpu.sync_copy(ones_hbm, ones_v)
        pltpu.sync_copy(z_hbm.at[pl.ds(sid * RPT, RPT)],
                        acc.at[pl.ds(sid * RPT, RPT)])
        plsc.subcore_barrier()

        def grp(i, carry):
            descs = []
            for k in range(GRP):
                descs.append(pltpu.async_copy(
                    ones_v, acc.at[idx_v.at[i * GRP + k]], sem, add=True))
            for d in descs:
                d.wait()
            return carry

        lax.fori_loop(0, NGRP, grp, 0)
        plsc.subcore_barrier()
        pltpu.sync_copy(acc.at[pl.ds(sid * RPT, RPT)],
                        out_hbm.at[cid, pl.ds(sid * RPT, RPT)])

    k = pl.kernel(
        body,
        out_type=jax.ShapeDtypeStruct((NC, NP, 16), jnp.float32),
        mesh=_mesh(),
        scratch_types=[
            pltpu.VMEM((NCHUNK, B), jnp.int32),
            pltpu.VMEM((B, 16), jnp.float32),
            pltpu.VMEM_SHARED((NP, 16), jnp.float32),
            pltpu.SemaphoreType.DMA,
        ],
        compiler_params=_SC_PARAMS,
    )
    return k(dst2d, ones, zeros16)


def _scatter_pass(g, src2d, dst2d, zeros, f):
    def body(g_hbm, src_hbm, dst_hbm, z_hbm, out_hbm,
             sidx_v, didx_v, rows_v, acc, gsem, ssem):
        cid = lax.axis_index("c")
        sid = lax.axis_index("s")
        wid = _wid()
        pltpu.sync_copy(src_hbm.at[pl.ds(wid * NCHUNK, NCHUNK)], sidx_v)
        pltpu.sync_copy(dst_hbm.at[pl.ds(wid * NCHUNK, NCHUNK)], didx_v)
        pltpu.sync_copy(z_hbm.at[pl.ds(sid * RPT, RPT)],
                        acc.at[pl.ds(sid * RPT, RPT)])
        plsc.subcore_barrier()

        def grp(i, carry):
            descs = []
            for k in range(GRP):
                descs.append(pltpu.async_copy(
                    g_hbm.at[sidx_v.at[i * GRP + k]], rows_v.at[k], gsem))
            for d in descs:
                d.wait()
            descs = []
            for k in range(GRP):
                descs.append(pltpu.async_copy(
                    rows_v.at[k], acc.at[didx_v.at[i * GRP + k]], ssem,
                    add=True))
            for d in descs:
                d.wait()
            return carry

        lax.fori_loop(0, NGRP, grp, 0)
        plsc.subcore_barrier()
        pltpu.sync_copy(acc.at[pl.ds(sid * RPT, RPT)],
                        out_hbm.at[cid, pl.ds(sid * RPT, RPT)])

    k = pl.kernel(
        body,
        out_type=jax.ShapeDtypeStruct((NC, NP, f), jnp.float32),
        mesh=_mesh(),
        scratch_types=[
            pltpu.VMEM((NCHUNK, B), jnp.int32),
            pltpu.VMEM((NCHUNK, B), jnp.int32),
            pltpu.VMEM((GRP, B, f), jnp.float32),
            pltpu.VMEM_SHARED((NP, f), jnp.float32),
            pltpu.SemaphoreType.DMA,
            pltpu.SemaphoreType.DMA,
        ],
        compiler_params=_SC_PARAMS,
    )
    return k(g, src2d, dst2d, zeros)


ROWS = 1024


def _dinv(d0_ref, d1_ref):
    deg = d0_ref[:, :1] + d1_ref[:, :1] + 1.0
    return lax.rsqrt(deg)


def _l1_body(x_ref, gl_ref, w_ref, d0_ref, d1_ref, o_ref):
    gw = jnp.dot(gl_ref[...], w_ref[...], preferred_element_type=jnp.float32)
    h = jnp.dot(x_ref[...], gw, preferred_element_type=jnp.float32)
    o_ref[...] = h * _dinv(d0_ref, d1_ref)


def _mid_body(s0_ref, s1_ref, g_ref, d0_ref, d1_ref, w_ref, b_ref, o_ref):
    dinv = _dinv(d0_ref, d1_ref)
    h = dinv * (s0_ref[...] + s1_ref[...] + g_ref[...]) + b_ref[0:1, :]
    h = jnp.maximum(h, 0.0)
    o_ref[...] = dinv * jnp.dot(h, w_ref[...],
                                preferred_element_type=jnp.float32)


def _fin_body(s0_ref, s1_ref, g_ref, d0_ref, d1_ref, b_ref, o_ref):
    dinv = _dinv(d0_ref, d1_ref)
    o = dinv * (s0_ref[...] + s1_ref[...] + g_ref[...]) + b_ref[0:1, :]
    m = jnp.max(o, axis=1, keepdims=True)
    lse = jnp.log(jnp.sum(jnp.exp(o - m), axis=1, keepdims=True)) + m
    o_ref[...] = o - lse


def _row_spec(f):
    return pl.BlockSpec((ROWS, f), lambda i: (i, 0))


def _full_spec(r, c):
    return pl.BlockSpec((r, c), lambda i: (0, 0))


def _tc_layer1(xp, glove, w1, d0, d1):
    return pl.pallas_call(
        _l1_body,
        grid=(NP // ROWS,),
        in_specs=[_row_spec(128), _full_spec(128, 128), _full_spec(128, 32),
                  _row_spec(16), _row_spec(16)],
        out_specs=_row_spec(32),
        out_shape=jax.ShapeDtypeStruct((NP, 32), jnp.float32),
    )(xp, glove, w1, d0, d1)


def _tc_mid(s0, s1, g, d0, d1, w, b, fin, fout):
    return pl.pallas_call(
        _mid_body,
        grid=(NP // ROWS,),
        in_specs=[_row_spec(fin), _row_spec(fin), _row_spec(fin),
                  _row_spec(16), _row_spec(16),
                  _full_spec(fin, fout), _full_spec(8, fin)],
        out_specs=_row_spec(fout),
        out_shape=jax.ShapeDtypeStruct((NP, fout), jnp.float32),
    )(s0, s1, g, d0, d1, w, b)


def _tc_final(s0, s1, g, d0, d1, b):
    return pl.pallas_call(
        _fin_body,
        grid=(NP // ROWS,),
        in_specs=[_row_spec(16), _row_spec(16), _row_spec(16),
                  _row_spec(16), _row_spec(16), _full_spec(8, 16)],
        out_specs=_row_spec(16),
        out_shape=jax.ShapeDtypeStruct((NP, 16), jnp.float32),
    )(s0, s1, g, d0, d1, b)


@jax.jit
def kernel(x, edge_index, glove, W1, b1, W2, b2, W3, b3):
    xp = jnp.pad(x, ((0, NP - N), (0, 0)))
    pad = jnp.full((EPAD - E,), NP - 1, dtype=jnp.int32)
    src2d = jnp.concatenate([edge_index[0], pad]).reshape(EPAD // B, B)
    dst2d = jnp.concatenate([edge_index[1], pad]).reshape(EPAD // B, B)
    ones = jnp.ones((B, 16), jnp.float32)
    z16 = jnp.zeros((NP, 16), jnp.float32)
    z32 = jnp.zeros((NP, 32), jnp.float32)
    b1b = jnp.broadcast_to(b1[None, :], (8, 32))
    b2b = jnp.broadcast_to(b2[None, :], (8, 32))
    b3b = jnp.broadcast_to(b3[None, :], (8, 16))

    deg = _degree_pass(dst2d, ones, z16)
    d0, d1 = deg[0], deg[1]

    g1 = _tc_layer1(xp, glove, W1, d0, d1)
    s1 = _scatter_pass(g1, src2d, dst2d, z32, 32)
    g2 = _tc_mid(s1[0], s1[1], g1, d0, d1, W2, b1b, 32, 32)
    s2 = _scatter_pass(g2, src2d, dst2d, z32, 32)
    g3 = _tc_mid(s2[0], s2[1], g2, d0, d1, W3, b2b, 32, 16)
    s3 = _scatter_pass(g3, src2d, dst2d, z16, 16)
    out = _tc_final(s3[0], s3[1], g3, d0, d1, b3b)
    return out[:N]


# speedup vs baseline: 24.4693x; 1.0631x over previous
import functools

import jax
import jax.numpy as jnp
from jax import lax
from jax.experimental import pallas as pl
from jax.experimental.pallas import tpu as pltpu
from jax.experimental.pallas import tpu_sc as plsc

N = 10000
NP = 10240
E = 320000
B = 128
NW = 32
EPAD = 2560 * B
NCHUNK = EPAD // B // NW
GRP = 8
NGRP = NCHUNK // GRP
RPT = NP // 16
NC = 2


def _mesh():
    return plsc.VectorSubcoreMesh(core_axis_name="c", subcore_axis_name="s")


_SC_PARAMS = pltpu.CompilerParams(use_tc_tiling_on_sc=False)


def _wid():
    return lax.axis_index("s") * NC + lax.axis_index("c")


def _degree_pass(dst2d, ones, zeros16):
    def body(dst_hbm, ones_hbm, z_hbm, out_hbm, idx_v, ones_v, acc, sem0,
             sem1):
        cid = lax.axis_index("c")
        sid = lax.axis_index("s")
        wid = _wid()
        pltpu.sync_copy(dst_hbm.at[pl.ds(wid * NCHUNK, NCHUNK)], idx_v)
        pltpu.sync_copy(ones_hbm, ones_v)
        pltpu.sync_copy(z_hbm.at[pl.ds(sid * RPT, RPT)],
                        acc.at[pl.ds(sid * RPT, RPT)])
        plsc.subcore_barrier()

        def s_issue(g, sem):
            return [pltpu.async_copy(ones_v, acc.at[idx_v.at[g * GRP + k]],
                                     sem, add=True) for k in range(GRP)]

        def s_wait(g, sem):
            for k in range(GRP):
                pltpu.make_async_copy(
                    ones_v, acc.at[idx_v.at[g * GRP + k]], sem).wait()

        s_issue(0, sem0)

        def pair(j, carry):
            s_issue(2 * j + 1, sem1)
            s_wait(2 * j, sem0)
            s_issue(2 * j + 2, sem0)
            s_wait(2 * j + 1, sem1)
            return carry

        lax.fori_loop(0, NGRP // 2 - 1, pair, 0)
        s_issue(NGRP - 1, sem1)
        s_wait(NGRP - 2, sem0)
        s_wait(NGRP - 1, sem1)
        plsc.subcore_barrier()
        pltpu.sync_copy(acc.at[pl.ds(sid * RPT, RPT)],
                        out_hbm.at[cid, pl.ds(sid * RPT, RPT)])

    k = pl.kernel(
        body,
        out_type=jax.ShapeDtypeStruct((NC, NP, 16), jnp.float32),
        mesh=_mesh(),
        scratch_types=[
            pltpu.VMEM((NCHUNK, B), jnp.int32),
            pltpu.VMEM((B, 16), jnp.float32),
            pltpu.VMEM_SHARED((NP, 16), jnp.float32),
            pltpu.SemaphoreType.DMA,
            pltpu.SemaphoreType.DMA,
        ],
        compiler_params=_SC_PARAMS,
    )
    return k(dst2d, ones, zeros16)


def _scatter_pass(g, src2d, dst2d, zeros, f):
    def body(g_hbm, src_hbm, dst_hbm, z_hbm, out_hbm,
             sidx_v, didx_v, rows_v, acc, gsem0, gsem1, ssem0, ssem1):
        cid = lax.axis_index("c")
        sid = lax.axis_index("s")
        wid = _wid()
        pltpu.sync_copy(src_hbm.at[pl.ds(wid * NCHUNK, NCHUNK)], sidx_v)
        pltpu.sync_copy(dst_hbm.at[pl.ds(wid * NCHUNK, NCHUNK)], didx_v)
        pltpu.sync_copy(z_hbm.at[pl.ds(sid * RPT, RPT)],
                        acc.at[pl.ds(sid * RPT, RPT)])
        plsc.subcore_barrier()

        gsem = (gsem0, gsem1)
        ssem = (ssem0, ssem1)

        def g_issue(g, b):
            for k in range(GRP):
                pltpu.async_copy(g_hbm.at[sidx_v.at[g * GRP + k]],
                                 rows_v.at[b, k], gsem[b])

        def g_wait(g, b):
            for k in range(GRP):
                pltpu.make_async_copy(g_hbm.at[sidx_v.at[g * GRP + k]],
                                      rows_v.at[b, k], gsem[b]).wait()

        def s_issue(g, b):
            for k in range(GRP):
                pltpu.async_copy(rows_v.at[b, k],
                                 acc.at[didx_v.at[g * GRP + k]], ssem[b],
                                 add=True)

        def s_wait(g, b):
            for k in range(GRP):
                pltpu.make_async_copy(rows_v.at[b, k],
                                      acc.at[didx_v.at[g * GRP + k]],
                                      ssem[b]).wait()

        g_issue(0, 0)

        def pair(j, carry):
            ga = 2 * j
            g_issue(ga + 1, 1)
            g_wait(ga, 0)
            s_issue(ga, 0)
            s_wait(ga, 0)
            g_issue(ga + 2, 0)
            g_wait(ga + 1, 1)
            s_issue(ga + 1, 1)
            s_wait(ga + 1, 1)
            return carry

        lax.fori_loop(0, NGRP // 2 - 1, pair, 0)
        g_issue(NGRP - 1, 1)
        g_wait(NGRP - 2, 0)
        s_issue(NGRP - 2, 0)
        s_wait(NGRP - 2, 0)
        g_wait(NGRP - 1, 1)
        s_issue(NGRP - 1, 1)
        s_wait(NGRP - 1, 1)
        plsc.subcore_barrier()
        pltpu.sync_copy(acc.at[pl.ds(sid * RPT, RPT)],
                        out_hbm.at[cid, pl.ds(sid * RPT, RPT)])

    k = pl.kernel(
        body,
        out_type=jax.ShapeDtypeStruct((NC, NP, f), jnp.float32),
        mesh=_mesh(),
        scratch_types=[
            pltpu.VMEM((NCHUNK, B), jnp.int32),
            pltpu.VMEM((NCHUNK, B), jnp.int32),
            pltpu.VMEM((2, GRP, B, f), jnp.float32),
            pltpu.VMEM_SHARED((NP, f), jnp.float32),
            pltpu.SemaphoreType.DMA,
            pltpu.SemaphoreType.DMA,
            pltpu.SemaphoreType.DMA,
            pltpu.SemaphoreType.DMA,
        ],
        compiler_params=_SC_PARAMS,
    )
    return k(g, src2d, dst2d, zeros)


ROWS = 1024


def _dinv(d0_ref, d1_ref):
    deg = d0_ref[:, :1] + d1_ref[:, :1] + 1.0
    return lax.rsqrt(deg)


def _l1_body(x_ref, gl_ref, w_ref, d0_ref, d1_ref, o_ref):
    gw = jnp.dot(gl_ref[...], w_ref[...], preferred_element_type=jnp.float32)
    h = jnp.dot(x_ref[...], gw, preferred_element_type=jnp.float32)
    o_ref[...] = h * _dinv(d0_ref, d1_ref)


def _mid_body(s0_ref, s1_ref, g_ref, d0_ref, d1_ref, w_ref, b_ref, o_ref):
    dinv = _dinv(d0_ref, d1_ref)
    h = dinv * (s0_ref[...] + s1_ref[...] + g_ref[...]) + b_ref[0:1, :]
    h = jnp.maximum(h, 0.0)
    o_ref[...] = dinv * jnp.dot(h, w_ref[...],
                                preferred_element_type=jnp.float32)


def _fin_body(s0_ref, s1_ref, g_ref, d0_ref, d1_ref, b_ref, o_ref):
    dinv = _dinv(d0_ref, d1_ref)
    o = dinv * (s0_ref[...] + s1_ref[...] + g_ref[...]) + b_ref[0:1, :]
    m = jnp.max(o, axis=1, keepdims=True)
    lse = jnp.log(jnp.sum(jnp.exp(o - m), axis=1, keepdims=True)) + m
    o_ref[...] = o - lse


def _row_spec(f):
    return pl.BlockSpec((ROWS, f), lambda i: (i, 0))


def _full_spec(r, c):
    return pl.BlockSpec((r, c), lambda i: (0, 0))


def _tc_layer1(xp, glove, w1, d0, d1):
    return pl.pallas_call(
        _l1_body,
        grid=(NP // ROWS,),
        in_specs=[_row_spec(128), _full_spec(128, 128), _full_spec(128, 32),
                  _row_spec(16), _row_spec(16)],
        out_specs=_row_spec(32),
        out_shape=jax.ShapeDtypeStruct((NP, 32), jnp.float32),
    )(xp, glove, w1, d0, d1)


def _tc_mid(s0, s1, g, d0, d1, w, b, fin, fout):
    return pl.pallas_call(
        _mid_body,
        grid=(NP // ROWS,),
        in_specs=[_row_spec(fin), _row_spec(fin), _row_spec(fin),
                  _row_spec(16), _row_spec(16),
                  _full_spec(fin, fout), _full_spec(8, fin)],
        out_specs=_row_spec(fout),
        out_shape=jax.ShapeDtypeStruct((NP, fout), jnp.float32),
    )(s0, s1, g, d0, d1, w, b)


def _tc_final(s0, s1, g, d0, d1, b):
    return pl.pallas_call(
        _fin_body,
        grid=(NP // ROWS,),
        in_specs=[_row_spec(16), _row_spec(16), _row_spec(16),
                  _row_spec(16), _row_spec(16), _full_spec(8, 16)],
        out_specs=_row_spec(16),
        out_shape=jax.ShapeDtypeStruct((NP, 16), jnp.float32),
    )(s0, s1, g, d0, d1, b)


@jax.jit
def kernel(x, edge_index, glove, W1, b1, W2, b2, W3, b3):
    xp = jnp.pad(x, ((0, NP - N), (0, 0)))
    pad = jnp.full((EPAD - E,), NP - 1, dtype=jnp.int32)
    src2d = jnp.concatenate([edge_index[0], pad]).reshape(EPAD // B, B)
    dst2d = jnp.concatenate([edge_index[1], pad]).reshape(EPAD // B, B)
    ones = jnp.ones((B, 16), jnp.float32)
    z16 = jnp.zeros((NP, 16), jnp.float32)
    z32 = jnp.zeros((NP, 32), jnp.float32)
    b1b = jnp.broadcast_to(b1[None, :], (8, 32))
    b2b = jnp.broadcast_to(b2[None, :], (8, 32))
    b3b = jnp.broadcast_to(b3[None, :], (8, 16))

    deg = _degree_pass(dst2d, ones, z16)
    d0, d1 = deg[0], deg[1]

    g1 = _tc_layer1(xp, glove, W1, d0, d1)
    s1 = _scatter_pass(g1, src2d, dst2d, z32, 32)
    g2 = _tc_mid(s1[0], s1[1], g1, d0, d1, W2, b1b, 32, 32)
    s2 = _scatter_pass(g2, src2d, dst2d, z32, 32)
    g3 = _tc_mid(s2[0], s2[1], g2, d0, d1, W3, b2b, 32, 16)
    s3 = _scatter_pass(g3, src2d, dst2d, z16, 16)
    out = _tc_final(s3[0], s3[1], g3, d0, d1, b3b)
    return out[:N]


# speedup vs baseline: 42.3468x; 1.7306x over previous
import functools

import jax
import jax.numpy as jnp
from jax import lax
from jax.experimental import pallas as pl
from jax.experimental.pallas import tpu as pltpu
from jax.experimental.pallas import tpu_sc as plsc

N = 10000
NP = 10240
E = 320000
B = 128
NW = 32
EPAD = 2560 * B
NCHUNK = EPAD // B // NW
GRP = 8
NGRP = NCHUNK // GRP
RPT = NP // 16
NC = 2


def _mesh():
    return plsc.VectorSubcoreMesh(core_axis_name="c", subcore_axis_name="s")


_SC_PARAMS = pltpu.CompilerParams(use_tc_tiling_on_sc=False)


def _wid():
    return lax.axis_index("s") * NC + lax.axis_index("c")


def _degree_pass(dst2d, ones, zeros16):
    def body(dst_hbm, ones_hbm, z_hbm, out_hbm, idx_v, ones_v, acc, sem0,
             sem1):
        cid = lax.axis_index("c")
        sid = lax.axis_index("s")
        wid = _wid()
        pltpu.sync_copy(dst_hbm.at[pl.ds(wid * NCHUNK, NCHUNK)], idx_v)
        pltpu.sync_copy(ones_hbm, ones_v)
        pltpu.sync_copy(z_hbm.at[pl.ds(sid * RPT, RPT)],
                        acc.at[pl.ds(sid * RPT, RPT)])
        plsc.subcore_barrier()

        def s_issue(g, sem):
            return [pltpu.async_copy(ones_v, acc.at[idx_v.at[g * GRP + k]],
                                     sem, add=True) for k in range(GRP)]

        def s_wait(g, sem):
            for k in range(GRP):
                pltpu.make_async_copy(
                    ones_v, acc.at[idx_v.at[g * GRP + k]], sem).wait()

        s_issue(0, sem0)

        def pair(j, carry):
            s_issue(2 * j + 1, sem1)
            s_wait(2 * j, sem0)
            s_issue(2 * j + 2, sem0)
            s_wait(2 * j + 1, sem1)
            return carry

        lax.fori_loop(0, NGRP // 2 - 1, pair, 0)
        s_issue(NGRP - 1, sem1)
        s_wait(NGRP - 2, sem0)
        s_wait(NGRP - 1, sem1)
        plsc.subcore_barrier()
        pltpu.sync_copy(acc.at[pl.ds(sid * RPT, RPT)],
                        out_hbm.at[cid, pl.ds(sid * RPT, RPT)])

    k = pl.kernel(
        body,
        out_type=jax.ShapeDtypeStruct((NC, NP, 16), jnp.float32),
        mesh=_mesh(),
        scratch_types=[
            pltpu.VMEM((NCHUNK, B), jnp.int32),
            pltpu.VMEM((B, 16), jnp.float32),
            pltpu.VMEM_SHARED((NP, 16), jnp.float32),
            pltpu.SemaphoreType.DMA,
            pltpu.SemaphoreType.DMA,
        ],
        compiler_params=_SC_PARAMS,
    )
    return k(dst2d, ones, zeros16)


def _scatter_pass(g, src2d, dst2d, zeros, f):
    def body(g_hbm, src_hbm, dst_hbm, z_hbm, out_hbm,
             sidx_v, didx_v, rows_v, acc, gsem0, gsem1, ssem0, ssem1):
        cid = lax.axis_index("c")
        sid = lax.axis_index("s")
        wid = _wid()
        pltpu.sync_copy(src_hbm.at[pl.ds(wid * NCHUNK, NCHUNK)], sidx_v)
        pltpu.sync_copy(dst_hbm.at[pl.ds(wid * NCHUNK, NCHUNK)], didx_v)
        pltpu.sync_copy(z_hbm.at[pl.ds(sid * RPT, RPT)],
                        acc.at[pl.ds(sid * RPT, RPT)])
        plsc.subcore_barrier()

        gsem = (gsem0, gsem1)
        ssem = (ssem0, ssem1)

        def g_issue(g, b):
            for k in range(GRP):
                pltpu.async_copy(g_hbm.at[sidx_v.at[g * GRP + k]],
                                 rows_v.at[b, k], gsem[b])

        def g_wait(g, b):
            for k in range(GRP):
                pltpu.make_async_copy(g_hbm.at[sidx_v.at[g * GRP + k]],
                                      rows_v.at[b, k], gsem[b]).wait()

        def s_issue(g, b):
            for k in range(GRP):
                pltpu.async_copy(rows_v.at[b, k],
                                 acc.at[didx_v.at[g * GRP + k]], ssem[b],
                                 add=True)

        def s_wait(g, b):
            for k in range(GRP):
                pltpu.make_async_copy(rows_v.at[b, k],
                                      acc.at[didx_v.at[g * GRP + k]],
                                      ssem[b]).wait()

        g_issue(0, 0)

        def pair(j, carry):
            ga = 2 * j
            g_issue(ga + 1, 1)
            g_wait(ga, 0)
            s_issue(ga, 0)
            s_wait(ga, 0)
            g_issue(ga + 2, 0)
            g_wait(ga + 1, 1)
            s_issue(ga + 1, 1)
            s_wait(ga + 1, 1)
            return carry

        lax.fori_loop(0, NGRP // 2 - 1, pair, 0)
        g_issue(NGRP - 1, 1)
        g_wait(NGRP - 2, 0)
        s_issue(NGRP - 2, 0)
        s_wait(NGRP - 2, 0)
        g_wait(NGRP - 1, 1)
        s_issue(NGRP - 1, 1)
        s_wait(NGRP - 1, 1)
        plsc.subcore_barrier()
        pltpu.sync_copy(acc.at[pl.ds(sid * RPT, RPT)],
                        out_hbm.at[cid, pl.ds(sid * RPT, RPT)])

    k = pl.kernel(
        body,
        out_type=jax.ShapeDtypeStruct((NC, NP, f), jnp.float32),
        mesh=_mesh(),
        scratch_types=[
            pltpu.VMEM((NCHUNK, B), jnp.int32),
            pltpu.VMEM((NCHUNK, B), jnp.int32),
            pltpu.VMEM((2, GRP, B, f), jnp.float32),
            pltpu.VMEM_SHARED((NP, f), jnp.float32),
            pltpu.SemaphoreType.DMA,
            pltpu.SemaphoreType.DMA,
            pltpu.SemaphoreType.DMA,
            pltpu.SemaphoreType.DMA,
        ],
        compiler_params=_SC_PARAMS,
    )
    return k(g, src2d, dst2d, zeros)


ROWS = 1024


def _dinv(d0_ref, d1_ref):
    deg = d0_ref[:, :1] + d1_ref[:, :1] + 1.0
    return lax.rsqrt(deg)


def _l1_body(x_ref, gl_ref, w_ref, d0_ref, d1_ref, o_ref):
    gw = jnp.dot(gl_ref[...], w_ref[...], preferred_element_type=jnp.float32)
    h = jnp.dot(x_ref[...], gw, preferred_element_type=jnp.float32)
    o_ref[...] = h * _dinv(d0_ref, d1_ref)


def _mid_body(s0_ref, s1_ref, g_ref, d0_ref, d1_ref, w_ref, b_ref, o_ref):
    dinv = _dinv(d0_ref, d1_ref)
    h = dinv * (s0_ref[...] + s1_ref[...] + g_ref[...]) + b_ref[0:1, :]
    h = jnp.maximum(h, 0.0)
    o_ref[...] = dinv * jnp.dot(h, w_ref[...],
                                preferred_element_type=jnp.float32)


def _fin_body(s0_ref, s1_ref, g_ref, d0_ref, d1_ref, b_ref, o_ref):
    dinv = _dinv(d0_ref, d1_ref)
    o = dinv * (s0_ref[...] + s1_ref[...] + g_ref[...]) + b_ref[0:1, :]
    m = jnp.max(o, axis=1, keepdims=True)
    lse = jnp.log(jnp.sum(jnp.exp(o - m), axis=1, keepdims=True)) + m
    o_ref[...] = o - lse


def _row_spec(f):
    return pl.BlockSpec((ROWS, f), lambda i: (i, 0))


def _full_spec(r, c):
    return pl.BlockSpec((r, c), lambda i: (0, 0))


def _tc_layer1(xp, glove, w1, d0, d1):
    return pl.pallas_call(
        _l1_body,
        grid=(NP // ROWS,),
        in_specs=[_row_spec(128), _full_spec(128, 128), _full_spec(128, 32),
                  _row_spec(16), _row_spec(16)],
        out_specs=_row_spec(32),
        out_shape=jax.ShapeDtypeStruct((NP, 32), jnp.float32),
    )(xp, glove, w1, d0, d1)


def _tc_mid(s0, s1, g, d0, d1, w, b, fin, fout):
    return pl.pallas_call(
        _mid_body,
        grid=(NP // ROWS,),
        in_specs=[_row_spec(fin), _row_spec(fin), _row_spec(fin),
                  _row_spec(16), _row_spec(16),
                  _full_spec(fin, fout), _full_spec(8, fin)],
        out_specs=_row_spec(fout),
        out_shape=jax.ShapeDtypeStruct((NP, fout), jnp.float32),
    )(s0, s1, g, d0, d1, w, b)


def _tc_final(s0, s1, g, d0, d1, b):
    return pl.pallas_call(
        _fin_body,
        grid=(NP // ROWS,),
        in_specs=[_row_spec(16), _row_spec(16), _row_spec(16),
                  _row_spec(16), _row_spec(16), _full_spec(8, 16)],
        out_specs=_row_spec(16),
        out_shape=jax.ShapeDtypeStruct((NP, 16), jnp.float32),
    )(s0, s1, g, d0, d1, b)


@jax.jit
def kernel(x, edge_index, glove, W1, b1, W2, b2, W3, b3):
    xp = jnp.pad(x, ((0, NP - N), (0, 0)))
    pad = N + (jnp.arange(EPAD - E, dtype=jnp.int32) % (NP - N))
    src2d = jnp.concatenate([edge_index[0], pad]).reshape(EPAD // B, B)
    dst2d = jnp.concatenate([edge_index[1], pad]).reshape(EPAD // B, B)
    ones = jnp.ones((B, 16), jnp.float32)
    z16 = jnp.zeros((NP, 16), jnp.float32)
    z32 = jnp.zeros((NP, 32), jnp.float32)
    b1b = jnp.broadcast_to(b1[None, :], (8, 32))
    b2b = jnp.broadcast_to(b2[None, :], (8, 32))
    b3b = jnp.broadcast_to(b3[None, :], (8, 16))

    deg = _degree_pass(dst2d, ones, z16)
    d0, d1 = deg[0], deg[1]

    g1 = _tc_layer1(xp, glove, W1, d0, d1)
    s1 = _scatter_pass(g1, src2d, dst2d, z32, 32)
    g2 = _tc_mid(s1[0], s1[1], g1, d0, d1, W2, b1b, 32, 32)
    s2 = _scatter_pass(g2, src2d, dst2d, z32, 32)
    g3 = _tc_mid(s2[0], s2[1], g2, d0, d1, W3, b2b, 32, 16)
    s3 = _scatter_pass(g3, src2d, dst2d, z16, 16)
    out = _tc_final(s3[0], s3[1], g3, d0, d1, b3b)
    return out[:N]


# speedup vs baseline: 55.7837x; 1.3173x over previous
import jax
import jax.numpy as jnp
from jax import lax
from jax.experimental import pallas as pl
from jax.experimental.pallas import tpu as pltpu
from jax.experimental.pallas import tpu_sc as plsc

N = 10000
NP = 10240
E = 320000
B = 128
NW = 32
EPAD = 2560 * B
NCHUNK = EPAD // B // NW
GRP = 8
NGRP = NCHUNK // GRP
RPT = NP // 16
NC = 2
PR = NP // 4
ROWS = 256
NPAD = NP - N


def _mesh():
    return plsc.VectorSubcoreMesh(core_axis_name="c", subcore_axis_name="s")


_SC_PARAMS = pltpu.CompilerParams(use_tc_tiling_on_sc=False)


def _wid():
    return lax.axis_index("s") * NC + lax.axis_index("c")


def _degree_pass(dst2d, ones, zeros32):
    def body(dst_hbm, ones_hbm, z_hbm, out_hbm, idx_v, ones_v, acc, sem0,
             sem1):
        cid = lax.axis_index("c")
        sid = lax.axis_index("s")
        wid = _wid()
        pltpu.sync_copy(dst_hbm.at[pl.ds(wid * NCHUNK, NCHUNK)], idx_v)
        pltpu.sync_copy(ones_hbm, ones_v)
        pltpu.sync_copy(z_hbm.at[pl.ds(sid * RPT, RPT)],
                        acc.at[pl.ds(sid * RPT, RPT)])
        plsc.subcore_barrier()

        def s_issue(g, sem):
            for k in range(GRP):
                pltpu.async_copy(ones_v, acc.at[idx_v.at[g * GRP + k]], sem,
                                 add=True)

        def s_wait(g, sem):
            for k in range(GRP):
                pltpu.make_async_copy(
                    ones_v, acc.at[idx_v.at[g * GRP + k]], sem).wait()

        s_issue(0, sem0)

        def pair(j, carry):
            s_issue(2 * j + 1, sem1)
            s_wait(2 * j, sem0)
            s_issue(2 * j + 2, sem0)
            s_wait(2 * j + 1, sem1)
            return carry

        lax.fori_loop(0, NGRP // 2 - 1, pair, 0)
        s_issue(NGRP - 1, sem1)
        s_wait(NGRP - 2, sem0)
        s_wait(NGRP - 1, sem1)
        plsc.subcore_barrier()
        pltpu.sync_copy(acc.at[pl.ds(sid * RPT, RPT)],
                        out_hbm.at[cid, pl.ds(sid * RPT, RPT)])

    k = pl.kernel(
        body,
        out_type=jax.ShapeDtypeStruct((NC, NP, 32), jnp.float32),
        mesh=_mesh(),
        scratch_types=[
            pltpu.VMEM((NCHUNK, B), jnp.int32),
            pltpu.VMEM((B, 32), jnp.float32),
            pltpu.VMEM_SHARED((NP, 32), jnp.float32),
            pltpu.SemaphoreType.DMA,
            pltpu.SemaphoreType.DMA,
        ],
        compiler_params=_SC_PARAMS,
    )
    return k(dst2d, ones, zeros32)


def _scatter_pass(g, src2d, dst2d, zeros):
    def body(g_hbm, src_hbm, dst_hbm, z_hbm, out_hbm,
             sidx_v, didx_v, rows_v, acc, gsem0, gsem1, ssem0, ssem1):
        cid = lax.axis_index("c")
        sid = lax.axis_index("s")
        wid = _wid()
        pltpu.sync_copy(src_hbm.at[pl.ds(wid * NCHUNK, NCHUNK)], sidx_v)
        pltpu.sync_copy(dst_hbm.at[pl.ds(wid * NCHUNK, NCHUNK)], didx_v)
        pltpu.sync_copy(z_hbm.at[pl.ds(sid * RPT, RPT)],
                        acc.at[pl.ds(sid * RPT, RPT)])
        plsc.subcore_barrier()

        gsem = (gsem0, gsem1)
        ssem = (ssem0, ssem1)

        def g_issue(g_, b):
            for k in range(GRP):
                pltpu.async_copy(g_hbm.at[sidx_v.at[g_ * GRP + k]],
                                 rows_v.at[b, k], gsem[b])

        def g_wait(g_, b):
            for k in range(GRP):
                pltpu.make_async_copy(g_hbm.at[sidx_v.at[g_ * GRP + k]],
                                      rows_v.at[b, k], gsem[b]).wait()

        def s_issue(g_, b):
            for k in range(GRP):
                pltpu.async_copy(rows_v.at[b, k],
                                 acc.at[didx_v.at[g_ * GRP + k]], ssem[b],
                                 add=True)

        def s_wait(g_, b):
            for k in range(GRP):
                pltpu.make_async_copy(rows_v.at[b, k],
                                      acc.at[didx_v.at[g_ * GRP + k]],
                                      ssem[b]).wait()

        g_issue(0, 0)

        def pair(j, carry):
            ga = 2 * j
            g_issue(ga + 1, 1)
            g_wait(ga, 0)
            s_issue(ga, 0)
            s_wait(ga, 0)
            g_issue(ga + 2, 0)
            g_wait(ga + 1, 1)
            s_issue(ga + 1, 1)
            s_wait(ga + 1, 1)
            return carry

        lax.fori_loop(0, NGRP // 2 - 1, pair, 0)
        g_issue(NGRP - 1, 1)
        g_wait(NGRP - 2, 0)
        s_issue(NGRP - 2, 0)
        s_wait(NGRP - 2, 0)
        g_wait(NGRP - 1, 1)
        s_issue(NGRP - 1, 1)
        s_wait(NGRP - 1, 1)
        plsc.subcore_barrier()
        pltpu.sync_copy(acc.at[pl.ds(sid * RPT, RPT)],
                        out_hbm.at[cid, pl.ds(sid * RPT, RPT)])

    k = pl.kernel(
        body,
        out_type=jax.ShapeDtypeStruct((NC, NP, 32), jnp.float32),
        mesh=_mesh(),
        scratch_types=[
            pltpu.VMEM((NCHUNK, B), jnp.int32),
            pltpu.VMEM((NCHUNK, B), jnp.int32),
            pltpu.VMEM((2, GRP, B, 32), jnp.float32),
            pltpu.VMEM_SHARED((NP, 32), jnp.float32),
            pltpu.SemaphoreType.DMA,
            pltpu.SemaphoreType.DMA,
            pltpu.SemaphoreType.DMA,
            pltpu.SemaphoreType.DMA,
        ],
        compiler_params=_SC_PARAMS,
    )
    return k(g, src2d, dst2d, zeros)


EB = 8192


def _edges_body(ei_ref, src_ref, dst_ref):
    i = pl.program_id(0)
    e = (i * EB + lax.broadcasted_iota(jnp.int32, (EB // B, B), 0) * B
         + lax.broadcasted_iota(jnp.int32, (EB // B, B), 1))
    padv = N + lax.rem(e - E, NPAD)
    src = jnp.reshape(ei_ref[0:1, :], (EB // B, B))
    dst = jnp.reshape(ei_ref[1:2, :], (EB // B, B))
    ok = e < E
    src_ref[...] = jnp.where(ok, src, padv)
    dst_ref[...] = jnp.where(ok, dst, padv)


def _prep_edges(edge_index):
    return pl.pallas_call(
        _edges_body,
        grid=(EPAD // EB,),
        in_specs=[pl.BlockSpec((2, EB), lambda i: (0, i))],
        out_specs=[pl.BlockSpec((EB // B, B), lambda i: (i, 0)),
                   pl.BlockSpec((EB // B, B), lambda i: (i, 0))],
        out_shape=[jax.ShapeDtypeStruct((EPAD // B, B), jnp.int32),
                   jax.ShapeDtypeStruct((EPAD // B, B), jnp.int32)],
    )(edge_index)


def _weights_body(gl_ref, w1_ref, w2_ref, w3_ref, w1p_ref, w2p_ref, w3p_ref):
    gw = jnp.dot(gl_ref[...], w1_ref[...], preferred_element_type=jnp.float32)
    w1p_ref[...] = jnp.zeros((512, 128), jnp.float32)
    w2p_ref[...] = jnp.zeros((128, 128), jnp.float32)
    w3p_ref[...] = jnp.zeros((128, 128), jnp.float32)
    for k in range(4):
        w1p_ref[pl.ds(k * 128, 128), pl.ds(k * 32, 32)] = gw
        w2p_ref[pl.ds(k * 32, 32), pl.ds(k * 32, 32)] = w2_ref[...]
        w3p_ref[pl.ds(k * 32, 32), pl.ds(k * 32, 16)] = w3_ref[...]


def _prep_weights(glove, w1, w2, w3):
    return pl.pallas_call(
        _weights_body,
        in_specs=[pl.BlockSpec((128, 128), lambda: (0, 0)),
                  pl.BlockSpec((128, 32), lambda: (0, 0)),
                  pl.BlockSpec((32, 32), lambda: (0, 0)),
                  pl.BlockSpec((32, 16), lambda: (0, 0))],
        out_specs=[pl.BlockSpec((512, 128), lambda: (0, 0)),
                   pl.BlockSpec((128, 128), lambda: (0, 0)),
                   pl.BlockSpec((128, 128), lambda: (0, 0))],
        out_shape=[jax.ShapeDtypeStruct((512, 128), jnp.float32),
                   jax.ShapeDtypeStruct((128, 128), jnp.float32),
                   jax.ShapeDtypeStruct((128, 128), jnp.float32)],
    )(glove, w1, w2, w3)


def _dinvp(d_ref):
    return lax.rsqrt(d_ref[0] + d_ref[1] + 1.0)


def _l1_body(x_ref, w_ref, d_ref, o_ref):
    h = jnp.dot(x_ref[...], w_ref[...], preferred_element_type=jnp.float32)
    o_ref[...] = h * _dinvp(d_ref)


def _mid_body(s_ref, g_ref, d_ref, w_ref, b_ref, o_ref):
    dinv = _dinvp(d_ref)
    h = dinv * (s_ref[0] + s_ref[1] + g_ref[...]) + b_ref[0:1, :]
    h = jnp.maximum(h, 0.0)
    o_ref[...] = dinv * jnp.dot(h, w_ref[...],
                                preferred_element_type=jnp.float32)


def _fin_body(s_ref, g_ref, d_ref, b_ref, o_ref):
    dinv = _dinvp(d_ref)
    o = dinv * (s_ref[0] + s_ref[1] + g_ref[...]) + b_ref[0:1, :]
    lane = lax.broadcasted_iota(jnp.int32, (ROWS, 128), 1)
    valid = lax.rem(lane, 32) < 16
    o = jnp.where(valid, o, -1e30)
    m = jnp.max(o, axis=1, keepdims=True)
    e = jnp.where(valid, jnp.exp(o - m), 0.0)
    ga = lax.broadcasted_iota(jnp.int32, (128, 128), 0) // 32
    gb = lax.broadcasted_iota(jnp.int32, (128, 128), 1) // 32
    mask = (ga == gb).astype(jnp.float32)
    s = jnp.dot(e, mask, preferred_element_type=jnp.float32)
    o_ref[...] = o - (jnp.log(s) + m)


def _row_spec():
    return pl.BlockSpec((ROWS, 128), lambda i: (i, 0))


def _pair_spec():
    return pl.BlockSpec((2, ROWS, 128), lambda i: (0, i, 0))


def _full(r, c):
    return pl.BlockSpec((r, c), lambda i: (0, 0))


def _tc_layer1(xp4, w1p, degp):
    return pl.pallas_call(
        _l1_body,
        grid=(PR // ROWS,),
        in_specs=[pl.BlockSpec((ROWS, 512), lambda i: (i, 0)),
                  _full(512, 128), _pair_spec()],
        out_specs=_row_spec(),
        out_shape=jax.ShapeDtypeStruct((PR, 128), jnp.float32),
    )(xp4, w1p, degp)


def _tc_mid(sp, gp, degp, wp, bp):
    return pl.pallas_call(
        _mid_body,
        grid=(PR // ROWS,),
        in_specs=[_pair_spec(), _row_spec(), _pair_spec(),
                  _full(128, 128), _full(8, 128)],
        out_specs=_row_spec(),
        out_shape=jax.ShapeDtypeStruct((PR, 128), jnp.float32),
    )(sp, gp, degp, wp, bp)


def _tc_final(sp, gp, degp, bp):
    return pl.pallas_call(
        _fin_body,
        grid=(PR // ROWS,),
        in_specs=[_pair_spec(), _row_spec(), _pair_spec(), _full(8, 128)],
        out_specs=_row_spec(),
        out_shape=jax.ShapeDtypeStruct((PR, 128), jnp.float32),
    )(sp, gp, degp, bp)


def _packed(a):
    return jnp.reshape(a, (NC, PR, 128))


def _table(p):
    return jnp.reshape(p, (NP, 32))


@jax.jit
def kernel(x, edge_index, glove, W1, b1, W2, b2, W3, b3):
    xp4 = jnp.pad(x, ((0, NPAD), (0, 0))).reshape(PR, 512)
    ones = jnp.ones((B, 32), jnp.float32)
    z32 = jnp.zeros((NP, 32), jnp.float32)
    b1p = jnp.broadcast_to(jnp.tile(b1, 4)[None, :], (8, 128))
    b2p = jnp.broadcast_to(jnp.tile(b2, 4)[None, :], (8, 128))
    b3p = jnp.broadcast_to(
        jnp.tile(jnp.pad(b3, (0, 16)), 4)[None, :], (8, 128))

    src2d, dst2d = _prep_edges(edge_index)
    w1p, w2p, w3p = _prep_weights(glove, W1, W2, W3)

    degp = _packed(_degree_pass(dst2d, ones, z32))

    g1 = _tc_layer1(xp4, w1p, degp)
    s1 = _packed(_scatter_pass(_table(g1), src2d, dst2d, z32))
    g2 = _tc_mid(s1, g1, degp, w2p, b1p)
    s2 = _packed(_scatter_pass(_table(g2), src2d, dst2d, z32))
    g3 = _tc_mid(s2, g2, degp, w3p, b2p)
    s3 = _packed(_scatter_pass(_table(g3), src2d, dst2d, z32))
    op = _tc_final(s3, g3, degp, b3p)
    return jnp.reshape(op, (NP, 32))[:N, :16]


# speedup vs baseline: 56.6780x; 1.0160x over previous
import jax
import jax.numpy as jnp
from jax import lax
from jax.experimental import pallas as pl
from jax.experimental.pallas import tpu as pltpu
from jax.experimental.pallas import tpu_sc as plsc

N = 10000
NP = 10240
E = 320000
B = 128
NW = 32
EPAD = 2560 * B
NCHUNK = EPAD // B // NW
GRP = 8
NGRP = NCHUNK // GRP
RPT = NP // 16
NC = 2
PR = NP // 4
ROWS = 256
NPAD = NP - N


def _mesh():
    return plsc.VectorSubcoreMesh(core_axis_name="c", subcore_axis_name="s")


_SC_PARAMS = pltpu.CompilerParams(use_tc_tiling_on_sc=False)


def _wid():
    return lax.axis_index("s") * NC + lax.axis_index("c")


def _degree_pass(dst2d, ones, zeros32):
    def body(dst_hbm, ones_hbm, z_hbm, out_hbm, idx_v, ones_v, acc, sem0,
             sem1):
        cid = lax.axis_index("c")
        sid = lax.axis_index("s")
        wid = _wid()
        pltpu.sync_copy(dst_hbm.at[pl.ds(wid * NCHUNK, NCHUNK)], idx_v)
        pltpu.sync_copy(ones_hbm, ones_v)
        pltpu.sync_copy(z_hbm.at[pl.ds(sid * RPT, RPT)],
                        acc.at[pl.ds(sid * RPT, RPT)])
        plsc.subcore_barrier()

        def s_issue(g, sem):
            for k in range(GRP):
                pltpu.async_copy(ones_v, acc.at[idx_v.at[g * GRP + k]], sem,
                                 add=True)

        def s_wait(g, sem):
            for k in range(GRP):
                pltpu.make_async_copy(
                    ones_v, acc.at[idx_v.at[g * GRP + k]], sem).wait()

        s_issue(0, sem0)

        def pair(j, carry):
            s_issue(2 * j + 1, sem1)
            s_wait(2 * j, sem0)
            s_issue(2 * j + 2, sem0)
            s_wait(2 * j + 1, sem1)
            return carry

        lax.fori_loop(0, NGRP // 2 - 1, pair, 0)
        s_issue(NGRP - 1, sem1)
        s_wait(NGRP - 2, sem0)
        s_wait(NGRP - 1, sem1)
        plsc.subcore_barrier()
        pltpu.sync_copy(acc.at[pl.ds(sid * RPT, RPT)],
                        out_hbm.at[cid, pl.ds(sid * RPT, RPT)])

    k = pl.kernel(
        body,
        out_type=jax.ShapeDtypeStruct((NC, NP, 32), jnp.float32),
        mesh=_mesh(),
        scratch_types=[
            pltpu.VMEM((NCHUNK, B), jnp.int32),
            pltpu.VMEM((B, 32), jnp.float32),
            pltpu.VMEM_SHARED((NP, 32), jnp.float32),
            pltpu.SemaphoreType.DMA,
            pltpu.SemaphoreType.DMA,
        ],
        compiler_params=_SC_PARAMS,
    )
    return k(dst2d, ones, zeros32)


def _scatter_pass(g, src2d, dst2d, zeros):
    def body(g_hbm, src_hbm, dst_hbm, z_hbm, out_hbm,
             sidx_v, didx_v, rows_v, acc, gsem0, gsem1, ssem0, ssem1):
        cid = lax.axis_index("c")
        sid = lax.axis_index("s")
        wid = _wid()
        pltpu.sync_copy(src_hbm.at[pl.ds(wid * NCHUNK, NCHUNK)], sidx_v)
        pltpu.sync_copy(dst_hbm.at[pl.ds(wid * NCHUNK, NCHUNK)], didx_v)
        pltpu.sync_copy(z_hbm.at[pl.ds(sid * RPT, RPT)],
                        acc.at[pl.ds(sid * RPT, RPT)])
        plsc.subcore_barrier()

        gsem = (gsem0, gsem1)
        ssem = (ssem0, ssem1)

        def g_issue(g_, b):
            for k in range(GRP):
                pltpu.async_copy(g_hbm.at[sidx_v.at[g_ * GRP + k]],
                                 rows_v.at[b, k], gsem[b])

        def g_wait(g_, b):
            for k in range(GRP):
                pltpu.make_async_copy(g_hbm.at[sidx_v.at[g_ * GRP + k]],
                                      rows_v.at[b, k], gsem[b]).wait()

        def s_issue(g_, b):
            for k in range(GRP):
                pltpu.async_copy(rows_v.at[b, k],
                                 acc.at[didx_v.at[g_ * GRP + k]], ssem[b],
                                 add=True)

        def s_wait(g_, b):
            for k in range(GRP):
                pltpu.make_async_copy(rows_v.at[b, k],
                                      acc.at[didx_v.at[g_ * GRP + k]],
                                      ssem[b]).wait()

        g_issue(0, 0)

        def pair(j, carry):
            ga = 2 * j
            g_issue(ga + 1, 1)
            g_wait(ga, 0)
            s_issue(ga, 0)
            s_wait(ga, 0)
            g_issue(ga + 2, 0)
            g_wait(ga + 1, 1)
            s_issue(ga + 1, 1)
            s_wait(ga + 1, 1)
            return carry

        lax.fori_loop(0, NGRP // 2 - 1, pair, 0)
        g_issue(NGRP - 1, 1)
        g_wait(NGRP - 2, 0)
        s_issue(NGRP - 2, 0)
        s_wait(NGRP - 2, 0)
        g_wait(NGRP - 1, 1)
        s_issue(NGRP - 1, 1)
        s_wait(NGRP - 1, 1)
        plsc.subcore_barrier()
        pltpu.sync_copy(acc.at[pl.ds(sid * RPT, RPT)],
                        out_hbm.at[cid, pl.ds(sid * RPT, RPT)])

    k = pl.kernel(
        body,
        out_type=jax.ShapeDtypeStruct((NC, NP, 32), jnp.float32),
        mesh=_mesh(),
        scratch_types=[
            pltpu.VMEM((NCHUNK, B), jnp.int32),
            pltpu.VMEM((NCHUNK, B), jnp.int32),
            pltpu.VMEM((2, GRP, B, 32), jnp.float32),
            pltpu.VMEM_SHARED((NP, 32), jnp.float32),
            pltpu.SemaphoreType.DMA,
            pltpu.SemaphoreType.DMA,
            pltpu.SemaphoreType.DMA,
            pltpu.SemaphoreType.DMA,
        ],
        compiler_params=_SC_PARAMS,
    )
    return k(g, src2d, dst2d, zeros)


EB = 8192


def _edges_body(ei_ref, src_ref, dst_ref):
    i = pl.program_id(0)
    e = (i * EB + lax.broadcasted_iota(jnp.int32, (EB // B, B), 0) * B
         + lax.broadcasted_iota(jnp.int32, (EB // B, B), 1))
    padv = N + lax.rem(e - E, NPAD)
    src = jnp.reshape(ei_ref[0:1, :], (EB // B, B))
    dst = jnp.reshape(ei_ref[1:2, :], (EB // B, B))
    ok = e < E
    src_ref[...] = jnp.where(ok, src, padv)
    dst_ref[...] = jnp.where(ok, dst, padv)


def _prep_edges(edge_index):
    return pl.pallas_call(
        _edges_body,
        grid=(EPAD // EB,),
        in_specs=[pl.BlockSpec((2, EB), lambda i: (0, i))],
        out_specs=[pl.BlockSpec((EB // B, B), lambda i: (i, 0)),
                   pl.BlockSpec((EB // B, B), lambda i: (i, 0))],
        out_shape=[jax.ShapeDtypeStruct((EPAD // B, B), jnp.int32),
                   jax.ShapeDtypeStruct((EPAD // B, B), jnp.int32)],
    )(edge_index)


def _weights_body(gl_ref, w1_ref, w2_ref, w3_ref, w1p_ref, w2p_ref, w3p_ref):
    gw = jnp.dot(gl_ref[...], w1_ref[...], preferred_element_type=jnp.float32)
    w1p_ref[...] = jnp.zeros((512, 128), jnp.float32)
    w2p_ref[...] = jnp.zeros((128, 128), jnp.float32)
    w3p_ref[...] = jnp.zeros((128, 128), jnp.float32)
    for k in range(4):
        w1p_ref[pl.ds(k * 128, 128), pl.ds(k * 32, 32)] = gw
        w2p_ref[pl.ds(k * 32, 32), pl.ds(k * 32, 32)] = w2_ref[...]
        w3p_ref[pl.ds(k * 32, 32), pl.ds(k * 32, 16)] = w3_ref[...]


def _prep_weights(glove, w1, w2, w3):
    return pl.pallas_call(
        _weights_body,
        in_specs=[pl.BlockSpec((128, 128), lambda: (0, 0)),
                  pl.BlockSpec((128, 32), lambda: (0, 0)),
                  pl.BlockSpec((32, 32), lambda: (0, 0)),
                  pl.BlockSpec((32, 16), lambda: (0, 0))],
        out_specs=[pl.BlockSpec((512, 128), lambda: (0, 0)),
                   pl.BlockSpec((128, 128), lambda: (0, 0)),
                   pl.BlockSpec((128, 128), lambda: (0, 0))],
        out_shape=[jax.ShapeDtypeStruct((512, 128), jnp.float32),
                   jax.ShapeDtypeStruct((128, 128), jnp.float32),
                   jax.ShapeDtypeStruct((128, 128), jnp.float32)],
    )(glove, w1, w2, w3)


def _dinvp(d_ref):
    return lax.rsqrt(d_ref[0] + d_ref[1] + 1.0)


def _mm_body(x_ref, w_ref, o_ref):
    o_ref[...] = jnp.dot(x_ref[...], w_ref[...],
                         preferred_element_type=jnp.float32)


def _scale_body(h_ref, d_ref, o_ref):
    o_ref[...] = h_ref[...] * _dinvp(d_ref)


def _mid_body(s_ref, g_ref, d_ref, w_ref, b_ref, o_ref):
    dinv = _dinvp(d_ref)
    h = dinv * (s_ref[0] + s_ref[1] + g_ref[...]) + b_ref[0:1, :]
    h = jnp.maximum(h, 0.0)
    o_ref[...] = dinv * jnp.dot(h, w_ref[...],
                                preferred_element_type=jnp.float32)


def _fin_body(s_ref, g_ref, d_ref, b_ref, o_ref):
    dinv = _dinvp(d_ref)
    o = dinv * (s_ref[0] + s_ref[1] + g_ref[...]) + b_ref[0:1, :]
    lane = lax.broadcasted_iota(jnp.int32, (ROWS, 128), 1)
    valid = lax.rem(lane, 32) < 16
    o = jnp.where(valid, o, -1e30)
    m = jnp.max(o, axis=1, keepdims=True)
    e = jnp.where(valid, jnp.exp(o - m), 0.0)
    ga = lax.broadcasted_iota(jnp.int32, (128, 128), 0) // 32
    gb = lax.broadcasted_iota(jnp.int32, (128, 128), 1) // 32
    mask = (ga == gb).astype(jnp.float32)
    s = jnp.dot(e, mask, preferred_element_type=jnp.float32)
    o_ref[...] = o - (jnp.log(s) + m)


def _row_spec():
    return pl.BlockSpec((ROWS, 128), lambda i: (i, 0))


def _pair_spec():
    return pl.BlockSpec((2, ROWS, 128), lambda i: (0, i, 0))


def _full(r, c):
    return pl.BlockSpec((r, c), lambda i: (0, 0))


def _tc_matmul1(xp4, w1p):
    return pl.pallas_call(
        _mm_body,
        grid=(PR // ROWS,),
        in_specs=[pl.BlockSpec((ROWS, 512), lambda i: (i, 0)),
                  _full(512, 128)],
        out_specs=_row_spec(),
        out_shape=jax.ShapeDtypeStruct((PR, 128), jnp.float32),
    )(xp4, w1p)


def _tc_scale(hp, degp):
    return pl.pallas_call(
        _scale_body,
        grid=(PR // ROWS,),
        in_specs=[_row_spec(), _pair_spec()],
        out_specs=_row_spec(),
        out_shape=jax.ShapeDtypeStruct((PR, 128), jnp.float32),
    )(hp, degp)


def _tc_mid(sp, gp, degp, wp, bp):
    return pl.pallas_call(
        _mid_body,
        grid=(PR // ROWS,),
        in_specs=[_pair_spec(), _row_spec(), _pair_spec(),
                  _full(128, 128), _full(8, 128)],
        out_specs=_row_spec(),
        out_shape=jax.ShapeDtypeStruct((PR, 128), jnp.float32),
    )(sp, gp, degp, wp, bp)


def _tc_final(sp, gp, degp, bp):
    return pl.pallas_call(
        _fin_body,
        grid=(PR // ROWS,),
        in_specs=[_pair_spec(), _row_spec(), _pair_spec(), _full(8, 128)],
        out_specs=_row_spec(),
        out_shape=jax.ShapeDtypeStruct((PR, 128), jnp.float32),
    )(sp, gp, degp, bp)


def _packed(a):
    return jnp.reshape(a, (NC, PR, 128))


def _table(p):
    return jnp.reshape(p, (NP, 32))


@jax.jit
def kernel(x, edge_index, glove, W1, b1, W2, b2, W3, b3):
    xp4 = jnp.pad(x, ((0, NPAD), (0, 0))).reshape(PR, 512)
    ones = jnp.ones((B, 32), jnp.float32)
    z32 = jnp.zeros((NP, 32), jnp.float32)
    b1p = jnp.broadcast_to(jnp.tile(b1, 4)[None, :], (8, 128))
    b2p = jnp.broadcast_to(jnp.tile(b2, 4)[None, :], (8, 128))
    b3p = jnp.broadcast_to(
        jnp.tile(jnp.pad(b3, (0, 16)), 4)[None, :], (8, 128))

    src2d, dst2d = _prep_edges(edge_index)
    w1p, w2p, w3p = _prep_weights(glove, W1, W2, W3)

    degp = _packed(_degree_pass(dst2d, ones, z32))
    h1 = _tc_matmul1(xp4, w1p)

    g1 = _tc_scale(h1, degp)
    s1 = _packed(_scatter_pass(_table(g1), src2d, dst2d, z32))
    g2 = _tc_mid(s1, g1, degp, w2p, b1p)
    s2 = _packed(_scatter_pass(_table(g2), src2d, dst2d, z32))
    g3 = _tc_mid(s2, g2, degp, w3p, b2p)
    s3 = _packed(_scatter_pass(_table(g3), src2d, dst2d, z32))
    op = _tc_final(s3, g3, degp, b3p)
    return jnp.reshape(op, (NP, 32))[:N, :16]
